# baseline (speedup 1.0000x reference)
import functools

import jax
import jax.numpy as jnp
from jax import lax
from jax.experimental import pallas as pl
from jax.experimental.pallas import tpu as pltpu
from jax.experimental.pallas import tpu_sc as plsc

L = 16
NC = 2
NS = 16
NW = NC * NS
K = 1024
NQ = 8
UNROLL = 4


def _pool_body(img_hbm, spx_hbm, partial_hbm, idx_v, ia0, ia1, ib0, ib1,
               acc_a0, acc_a1, acc_b0, acc_b1, red_a, red_b,
               sa0, sa1, sb0, sb1):
    B, C, NPIX = img_hbm.shape
    ranges_per_batch = NW // B
    npix_t = NPIX // ranges_per_batch
    q_pix = npix_t // NQ

    cid = lax.axis_index("c")
    sid = lax.axis_index("s")
    wid = sid * NC + cid
    b = wid // ranges_per_batch
    r = wid % ranges_per_batch
    base = r * npix_t

    img_bufs = (ia0, ia1), (ib0, ib1)
    sems = (sa0, sa1), (sb0, sb1)

    neg = jnp.full((L,), -jnp.inf, jnp.float32)
    lane_off = lax.iota(jnp.int32, L) * K

    pltpu.sync_copy(spx_hbm.at[b, pl.ds(base, npix_t)], idx_v)

    @plsc.parallel_loop(0, L * K // L, unroll=8)
    def _init(i):
        acc_a0[pl.ds(i * L, L)] = neg
        acc_a1[pl.ds(i * L, L)] = neg
        acc_b0[pl.ds(i * L, L)] = neg
        acc_b1[pl.ds(i * L, L)] = neg

    @plsc.parallel_loop(0, npix_t // L, unroll=8)
    def _flatten(i):
        idx_v[pl.ds(i * L, L)] = idx_v[pl.ds(i * L, L)] + lane_off

    def _start_q(c0, q, buf_par):
        for ch in range(2):
            pltpu.async_copy(
                img_hbm.at[b, c0 + ch, pl.ds(base + q * q_pix, q_pix)],
                img_bufs[ch][buf_par], sems[ch][buf_par])

    def _wait_q(c0, q, buf_par):
        for ch in range(2):
            pltpu.make_async_copy(
                img_hbm.at[b, c0 + ch, pl.ds(base + q * q_pix, q_pix)],
                img_bufs[ch][buf_par], sems[ch][buf_par]).wait()

    _start_q(0, 0, 0)

    def _pair(p, carry):
        c0 = 2 * p
        for q in range(NQ):
            par = q % 2
            _wait_q(c0, q, par)
            if q + 1 < NQ:
                _start_q(c0, q + 1, (q + 1) % 2)
            if q == NQ - 1:
                @pl.when(p + 1 < C // 2)
                def _():
                    _start_q(c0 + 2, 0, 0)
            ia, ib = img_bufs[0][par], img_bufs[1][par]

            def _update(i, carry2):
                for u in range(UNROLL):
                    off = (i * UNROLL + u) * L
                    aa = acc_a0 if u % 2 == 0 else acc_a1
                    ab = acc_b0 if u % 2 == 0 else acc_b1
                    fidx = idx_v[pl.ds(q * q_pix + off, L)]
                    va = ia[pl.ds(off, L)]
                    vb = ib[pl.ds(off, L)]
                    oa = plsc.load_gather(aa, [fidx])
                    plsc.store_scatter(aa, [fidx], jnp.maximum(oa, va))
                    ob = plsc.load_gather(ab, [fidx])
                    plsc.store_scatter(ab, [fidx], jnp.maximum(ob, vb))
                return carry2

            lax.fori_loop(0, q_pix // L // UNROLL, _update, 0)

        @plsc.parallel_loop(0, K // L, unroll=2)
        def _reduce(g):
            ma = acc_a0[pl.ds(g * L, L)]
            mb = acc_b0[pl.ds(g * L, L)]
            acc_a0[pl.ds(g * L, L)] = neg
            acc_b0[pl.ds(g * L, L)] = neg
            for l in range(L):
                for acc, is_a in ((acc_a1, True), (acc_b1, False)):
                    off = l * K + g * L
                    v = acc[pl.ds(off, L)]
                    acc[pl.ds(off, L)] = neg
                    if is_a:
                        ma = jnp.maximum(ma, v)
                    else:
                        mb = jnp.maximum(mb, v)
            for l in range(1, L):
                off = l * K + g * L
                ma = jnp.maximum(ma, acc_a0[pl.ds(off, L)])
                mb = jnp.maximum(mb, acc_b0[pl.ds(off, L)])
                acc_a0[pl.ds(off, L)] = neg
                acc_b0[pl.ds(off, L)] = neg
            red_a[pl.ds(g * L, L)] = ma
            red_b[pl.ds(g * L, L)] = mb

        pltpu.sync_copy(red_a, partial_hbm.at[b, r, c0])
        pltpu.sync_copy(red_b, partial_hbm.at[b, r, c0 + 1])
        return carry

    lax.fori_loop(0, C // 2, _pair, 0)


def _merge_body(p_ref, o_ref):
    o_ref[...] = jnp.max(p_ref[...], axis=1)


@jax.jit
def kernel(img, spx):
    B, C, H, W = img.shape
    npix = H * W
    img3 = img.reshape(B, C, npix)
    spx2 = spx.reshape(B, npix)
    ranges_per_batch = NW // B

    mesh = plsc.VectorSubcoreMesh(
        core_axis_name="c", subcore_axis_name="s", num_cores=NC,
        num_subcores=NS)
    npix_t = npix // ranges_per_batch
    q_pix = npix_t // NQ
    pool = pl.kernel(
        _pool_body,
        out_type=jax.ShapeDtypeStruct((B, ranges_per_batch, C, K),
                                      jnp.float32),
        mesh=mesh,
        compiler_params=pltpu.CompilerParams(needs_layout_passes=False),
        scratch_types=[
            pltpu.VMEM((npix_t,), jnp.int32),
            pltpu.VMEM((q_pix,), jnp.float32),
            pltpu.VMEM((q_pix,), jnp.float32),
            pltpu.VMEM((q_pix,), jnp.float32),
            pltpu.VMEM((q_pix,), jnp.float32),
            pltpu.VMEM((L * K,), jnp.float32),
            pltpu.VMEM((L * K,), jnp.float32),
            pltpu.VMEM((L * K,), jnp.float32),
            pltpu.VMEM((L * K,), jnp.float32),
            pltpu.VMEM((K,), jnp.float32),
            pltpu.VMEM((K,), jnp.float32),
            pltpu.SemaphoreType.DMA,
            pltpu.SemaphoreType.DMA,
            pltpu.SemaphoreType.DMA,
            pltpu.SemaphoreType.DMA,
        ],
    )
    partial = pool(img3, spx2)

    out = pl.pallas_call(
        _merge_body,
        grid=(B,),
        in_specs=[pl.BlockSpec((1, ranges_per_batch, C, K),
                               lambda i: (i, 0, 0, 0))],
        out_specs=pl.BlockSpec((1, C, K), lambda i: (i, 0, 0)),
        out_shape=jax.ShapeDtypeStruct((B, C, K), jnp.float32),
    )(partial)
    return out


# speedup vs baseline: 1.7244x; 1.7244x over previous
import jax
import jax.numpy as jnp
from jax import lax
from jax.experimental import pallas as pl
from jax.experimental.pallas import tpu as pltpu
from jax.experimental.pallas import tpu_sc as plsc

L = 16
NC = 2
NS = 16
NW = NC * NS
K = 1024
NCH = 4
QPIX = 2048
UNROLL = 4


def _pool_body(img_hbm, spx_hbm, partial_hbm, idx_v,
               i00, i01, i10, i11, i20, i21, i30, i31,
               a0, a1, a2, a3, r0, r1, r2, r3,
               s00, s01, s10, s11, s20, s21, s30, s31):
    B, C, NPIX = img_hbm.shape
    ranges_per_batch = NW // B
    npix_t = NPIX // ranges_per_batch
    nchunk = npix_t // QPIX

    cid = lax.axis_index("c")
    sid = lax.axis_index("s")
    wid = sid * NC + cid
    b = wid // ranges_per_batch
    r = wid % ranges_per_batch
    base = r * npix_t

    bufs = ((i00, i01), (i10, i11), (i20, i21), (i30, i31))
    sems = ((s00, s01), (s10, s11), (s20, s21), (s30, s31))
    accs = (a0, a1, a2, a3)
    reds = (r0, r1, r2, r3)

    neg = jnp.full((L,), -jnp.inf, jnp.float32)
    lane_off = lax.iota(jnp.int32, L) * K

    pltpu.sync_copy(spx_hbm.at[b, pl.ds(base, npix_t)], idx_v)

    @plsc.parallel_loop(0, L * K // L, unroll=8)
    def _init(i):
        for ch in range(NCH):
            accs[ch][pl.ds(i * L, L)] = neg

    @plsc.parallel_loop(0, npix_t // L, unroll=8)
    def _flatten(i):
        idx_v[pl.ds(i * L, L)] = idx_v[pl.ds(i * L, L)] + lane_off

    def _start(c0, chunk, par):
        for ch in range(NCH):
            pltpu.async_copy(
                img_hbm.at[b, c0 + ch, pl.ds(base + chunk * QPIX, QPIX)],
                bufs[ch][par], sems[ch][par])

    def _wait(c0, chunk, par):
        for ch in range(NCH):
            pltpu.make_async_copy(
                img_hbm.at[b, c0 + ch, pl.ds(base + chunk * QPIX, QPIX)],
                bufs[ch][par], sems[ch][par]).wait()

    def _update_chunk(chunk, par):
        ia = tuple(bufs[ch][par] for ch in range(NCH))

        def _ub(i, carry):
            idxs = []
            vals = []
            for u in range(UNROLL):
                off = (i * UNROLL + u) * L
                idxs.append(idx_v[pl.ds(chunk * QPIX + off, L)])
            for u in range(UNROLL):
                off = (i * UNROLL + u) * L
                vals.append(tuple(ia[ch][pl.ds(off, L)]
                                  for ch in range(NCH)))
            for u in range(UNROLL):
                g = [plsc.load_gather(accs[ch], [idxs[u]])
                     for ch in range(NCH)]
                m = [jnp.maximum(g[ch], vals[u][ch]) for ch in range(NCH)]
                for ch in range(NCH):
                    plsc.store_scatter(accs[ch], [idxs[u]], m[ch])
            return carry

        lax.fori_loop(0, QPIX // L // UNROLL, _ub, 0)

    _start(0, 0, 0)

    def _group(grp, carry):
        c0 = NCH * grp

        def _chunkpair(j, carry2):
            ch0 = 2 * j
            _wait(c0, ch0, 0)
            _start(c0, ch0 + 1, 1)
            _update_chunk(ch0, 0)
            _wait(c0, ch0 + 1, 1)

            @pl.when(j + 1 < nchunk // 2)
            def _():
                _start(c0, ch0 + 2, 0)

            @pl.when(jnp.logical_and(j + 1 == nchunk // 2,
                                     grp + 1 < C // NCH))
            def _():
                _start(c0 + NCH, 0, 0)

            _update_chunk(ch0 + 1, 1)
            return carry2

        lax.fori_loop(0, nchunk // 2, _chunkpair, 0)

        @plsc.parallel_loop(0, K // L, unroll=2)
        def _reduce(g):
            for ch in range(NCH):
                m = accs[ch][pl.ds(g * L, L)]
                accs[ch][pl.ds(g * L, L)] = neg
                for l in range(1, L):
                    off = l * K + g * L
                    m = jnp.maximum(m, accs[ch][pl.ds(off, L)])
                    accs[ch][pl.ds(off, L)] = neg
                reds[ch][pl.ds(g * L, L)] = m

        for ch in range(NCH):
            pltpu.sync_copy(reds[ch], partial_hbm.at[b, r, c0 + ch])
        return carry

    lax.fori_loop(0, C // NCH, _group, 0)


def _merge_body(p_ref, o_ref):
    o_ref[...] = jnp.max(p_ref[...], axis=1)


@jax.jit
def kernel(img, spx):
    B, C, H, W = img.shape
    npix = H * W
    img3 = img.reshape(B, C, npix)
    spx2 = spx.reshape(B, npix)
    ranges_per_batch = NW // B

    mesh = plsc.VectorSubcoreMesh(
        core_axis_name="c", subcore_axis_name="s", num_cores=NC,
        num_subcores=NS)
    npix_t = npix // ranges_per_batch
    pool = pl.kernel(
        _pool_body,
        out_type=jax.ShapeDtypeStruct((B, ranges_per_batch, C, K),
                                      jnp.float32),
        mesh=mesh,
        compiler_params=pltpu.CompilerParams(needs_layout_passes=False),
        scratch_types=(
            [pltpu.VMEM((npix_t,), jnp.int32)]
            + [pltpu.VMEM((QPIX,), jnp.float32) for _ in range(2 * NCH)]
            + [pltpu.VMEM((L * K,), jnp.float32) for _ in range(NCH)]
            + [pltpu.VMEM((K,), jnp.float32) for _ in range(NCH)]
            + [pltpu.SemaphoreType.DMA for _ in range(2 * NCH)]
        ),
    )
    partial = pool(img3, spx2)

    out = pl.pallas_call(
        _merge_body,
        grid=(B,),
        in_specs=[pl.BlockSpec((1, ranges_per_batch, C, K),
                               lambda i: (i, 0, 0, 0))],
        out_specs=pl.BlockSpec((1, C, K), lambda i: (i, 0, 0)),
        out_shape=jax.ShapeDtypeStruct((B, C, K), jnp.float32),
    )(partial)
    return out


# speedup vs baseline: 2.1166x; 1.2274x over previous
import jax
import jax.numpy as jnp
from jax import lax
from jax.experimental import pallas as pl
from jax.experimental.pallas import tpu as pltpu
from jax.experimental.pallas import tpu_sc as plsc

L = 16
NC = 2
NS = 16
NW = NC * NS
K = 1024
NCH = 4
QPIX = 2048
UNROLL = 4


def _pool_body(C, img_hbm, spx_hbm, partial_hbm, idx_v,
               i00, i01, i10, i11, i20, i21, i30, i31,
               a0, a1, a2, a3, r0, r1, r2, r3,
               s00, s01, s10, s11, s20, s21, s30, s31):
    NPIX = spx_hbm.shape[0] // 4
    B = 4
    ranges_per_batch = NW // B
    npix_t = NPIX // ranges_per_batch
    nchunk = npix_t // QPIX

    cid = lax.axis_index("c")
    sid = lax.axis_index("s")
    wid = sid * NC + cid
    b = wid // ranges_per_batch
    r = wid % ranges_per_batch
    base = r * npix_t

    bufs = ((i00, i01), (i10, i11), (i20, i21), (i30, i31))
    sems = ((s00, s01), (s10, s11), (s20, s21), (s30, s31))
    accs = (a0, a1, a2, a3)
    reds = (r0, r1, r2, r3)

    neg = jnp.full((L,), -jnp.inf, jnp.float32)
    lane_off = lax.iota(jnp.int32, L) * K

    pltpu.sync_copy(spx_hbm.at[pl.ds(b * NPIX + base, npix_t)], idx_v)

    @plsc.parallel_loop(0, L * K // L, unroll=8)
    def _init(i):
        for ch in range(NCH):
            accs[ch][pl.ds(i * L, L)] = neg

    @plsc.parallel_loop(0, npix_t // L, unroll=8)
    def _flatten(i):
        idx_v[pl.ds(i * L, L)] = idx_v[pl.ds(i * L, L)] + lane_off

    def _img_off(c, chunk):
        return (b * C + c) * NPIX + base + chunk * QPIX

    def _start(c0, chunk, par):
        for ch in range(NCH):
            pltpu.async_copy(
                img_hbm.at[pl.ds(_img_off(c0 + ch, chunk), QPIX)],
                bufs[ch][par], sems[ch][par])

    def _wait(c0, chunk, par):
        for ch in range(NCH):
            pltpu.make_async_copy(
                img_hbm.at[pl.ds(_img_off(c0 + ch, chunk), QPIX)],
                bufs[ch][par], sems[ch][par]).wait()

    def _update_chunk(chunk, par):
        ia = tuple(bufs[ch][par] for ch in range(NCH))

        def _ub(i, carry):
            idxs = []
            vals = []
            for u in range(UNROLL):
                off = (i * UNROLL + u) * L
                idxs.append(idx_v[pl.ds(chunk * QPIX + off, L)])
            for u in range(UNROLL):
                off = (i * UNROLL + u) * L
                vals.append(tuple(ia[ch][pl.ds(off, L)]
                                  for ch in range(NCH)))
            for u in range(UNROLL):
                g = [plsc.load_gather(accs[ch], [idxs[u]])
                     for ch in range(NCH)]
                m = [jnp.maximum(g[ch], vals[u][ch]) for ch in range(NCH)]
                for ch in range(NCH):
                    plsc.store_scatter(accs[ch], [idxs[u]], m[ch])
            return carry

        lax.fori_loop(0, QPIX // L // UNROLL, _ub, 0)

    _start(0, 0, 0)

    def _group(grp, carry):
        c0 = NCH * grp

        def _chunkpair(j, carry2):
            ch0 = 2 * j
            _wait(c0, ch0, 0)
            _start(c0, ch0 + 1, 1)
            _update_chunk(ch0, 0)
            _wait(c0, ch0 + 1, 1)

            @pl.when(j + 1 < nchunk // 2)
            def _():
                _start(c0, ch0 + 2, 0)

            @pl.when(jnp.logical_and(j + 1 == nchunk // 2,
                                     grp + 1 < C // NCH))
            def _():
                _start(c0 + NCH, 0, 0)

            _update_chunk(ch0 + 1, 1)
            return carry2

        lax.fori_loop(0, nchunk // 2, _chunkpair, 0)

        @plsc.parallel_loop(0, K // L, unroll=2)
        def _reduce(g):
            for ch in range(NCH):
                m = accs[ch][pl.ds(g * L, L)]
                accs[ch][pl.ds(g * L, L)] = neg
                for l in range(1, L):
                    off = l * K + g * L
                    m = jnp.maximum(m, accs[ch][pl.ds(off, L)])
                    accs[ch][pl.ds(off, L)] = neg
                reds[ch][pl.ds(g * L, L)] = m

        for ch in range(NCH):
            pltpu.sync_copy(
                reds[ch],
                partial_hbm.at[pl.ds(
                    ((b * ranges_per_batch + r) * C + c0 + ch) * K, K)])
        return carry

    lax.fori_loop(0, C // NCH, _group, 0)


def _merge_body(p_ref, o_ref):
    o_ref[...] = jnp.max(p_ref[...], axis=1)


@jax.jit
def kernel(img, spx):
    B, C, H, W = img.shape
    npix = H * W
    img1 = img.reshape(B * C * npix)
    spx1 = spx.reshape(B * npix)
    ranges_per_batch = NW // B

    mesh = plsc.VectorSubcoreMesh(
        core_axis_name="c", subcore_axis_name="s", num_cores=NC,
        num_subcores=NS)
    npix_t = npix // ranges_per_batch
    import functools
    pool = pl.kernel(
        functools.partial(_pool_body, C),
        out_type=jax.ShapeDtypeStruct((B * ranges_per_batch * C * K,),
                                      jnp.float32),
        mesh=mesh,
        compiler_params=pltpu.CompilerParams(needs_layout_passes=False),
        scratch_types=(
            [pltpu.VMEM((npix_t,), jnp.int32)]
            + [pltpu.VMEM((QPIX,), jnp.float32) for _ in range(2 * NCH)]
            + [pltpu.VMEM((L * K,), jnp.float32) for _ in range(NCH)]
            + [pltpu.VMEM((K,), jnp.float32) for _ in range(NCH)]
            + [pltpu.SemaphoreType.DMA for _ in range(2 * NCH)]
        ),
    )
    partial = pool(img1, spx1).reshape(B, ranges_per_batch, C, K)

    out = pl.pallas_call(
        _merge_body,
        grid=(B,),
        in_specs=[pl.BlockSpec((1, ranges_per_batch, C, K),
                               lambda i: (i, 0, 0, 0))],
        out_specs=pl.BlockSpec((1, C, K), lambda i: (i, 0, 0)),
        out_shape=jax.ShapeDtypeStruct((B, C, K), jnp.float32),
    )(partial)
    return out


# speedup vs baseline: 2.8600x; 1.3513x over previous
import functools

import jax
import jax.numpy as jnp
from jax import lax
from jax.experimental import pallas as pl
from jax.experimental.pallas import tpu as pltpu
from jax.experimental.pallas import tpu_sc as plsc

L = 16
NC = 2
NS = 16
NW = NC * NS
K = 1024
NCH = 3
SROWS = 8
UNROLL = 4


def _pool_body(B, C, img_hbm, spx_hbm, partial_hbm, idx_v,
               i00, i01, i10, i11, i20, i21,
               a0, a1, a2, r0, r1, r2,
               s00, s01, s10, s11, s20, s21):
    _, H, W = spx_hbm.shape
    ranges_per_batch = NW // B
    rows_t = H // ranges_per_batch
    nstrip = rows_t // SROWS
    vec_per_row = W // L
    quads_per_row = vec_per_row // UNROLL

    cid = lax.axis_index("c")
    sid = lax.axis_index("s")
    wid = sid * NC + cid
    b = wid // ranges_per_batch
    r = wid % ranges_per_batch
    row0 = r * rows_t

    bufs = ((i00, i01), (i10, i11), (i20, i21))
    sems = ((s00, s01), (s10, s11), (s20, s21))
    accs = (a0, a1, a2)
    reds = (r0, r1, r2)

    neg = jnp.full((L,), -jnp.inf, jnp.float32)
    lane_off = lax.iota(jnp.int32, L) * K

    pltpu.sync_copy(spx_hbm.at[b, pl.ds(row0, rows_t)], idx_v)

    @plsc.parallel_loop(0, L * K // L, unroll=8)
    def _init(i):
        for ch in range(NCH):
            accs[ch][pl.ds(i * L, L)] = neg

    @plsc.parallel_loop(0, rows_t * vec_per_row, unroll=4)
    def _flatten(i):
        row = i >> 5
        col = (i & (vec_per_row - 1)) * L
        idx_v[row, pl.ds(col, L)] = idx_v[row, pl.ds(col, L)] + lane_off

    def _start(c0, strip, par):
        for ch in range(NCH):
            pltpu.async_copy(
                img_hbm.at[b * C + c0 + ch,
                           pl.ds(row0 + strip * SROWS, SROWS)],
                bufs[ch][par], sems[ch][par])

    def _wait(c0, strip, par):
        for ch in range(NCH):
            pltpu.make_async_copy(
                img_hbm.at[b * C + c0 + ch,
                           pl.ds(row0 + strip * SROWS, SROWS)],
                bufs[ch][par], sems[ch][par]).wait()

    def _update_strip(strip, par):
        ia = tuple(bufs[ch][par] for ch in range(NCH))

        def _ub(i, carry):
            row = i // quads_per_row
            co = (i % quads_per_row) * (UNROLL * L)
            srow = strip * SROWS + row
            idxs = []
            vals = []
            for u in range(UNROLL):
                idxs.append(idx_v[srow, pl.ds(co + u * L, L)])
            for u in range(UNROLL):
                vals.append(tuple(ia[ch][row, pl.ds(co + u * L, L)]
                                  for ch in range(NCH)))
            for u in range(UNROLL):
                g = [plsc.load_gather(accs[ch], [idxs[u]])
                     for ch in range(NCH)]
                m = [jnp.maximum(g[ch], vals[u][ch]) for ch in range(NCH)]
                for ch in range(NCH):
                    plsc.store_scatter(accs[ch], [idxs[u]], m[ch])
            return carry

        lax.fori_loop(0, SROWS * quads_per_row, _ub, 0)

    _start(0, 0, 0)

    def _group(grp, carry):
        c0 = NCH * grp

        def _strippair(j, carry2):
            st0 = 2 * j
            _wait(c0, st0, 0)
            _start(c0, st0 + 1, 1)
            _update_strip(st0, 0)
            _wait(c0, st0 + 1, 1)

            @pl.when(j + 1 < nstrip // 2)
            def _():
                _start(c0, st0 + 2, 0)

            @pl.when(jnp.logical_and(j + 1 == nstrip // 2,
                                     grp + 1 < C // NCH))
            def _():
                _start(c0 + NCH, 0, 0)

            _update_strip(st0 + 1, 1)
            return carry2

        lax.fori_loop(0, nstrip // 2, _strippair, 0)

        @plsc.parallel_loop(0, K // L, unroll=2)
        def _reduce(g):
            for ch in range(NCH):
                m = accs[ch][pl.ds(g * L, L)]
                accs[ch][pl.ds(g * L, L)] = neg
                for l in range(1, L):
                    off = l * K + g * L
                    m = jnp.maximum(m, accs[ch][pl.ds(off, L)])
                    accs[ch][pl.ds(off, L)] = neg
                reds[ch][pl.ds(g * L, L)] = m

        for ch in range(NCH):
            pltpu.sync_copy(
                reds[ch],
                partial_hbm.at[pl.ds(
                    ((b * ranges_per_batch + r) * C + c0 + ch) * K, K)])
        return carry

    lax.fori_loop(0, C // NCH, _group, 0)


def _merge_body(p_ref, o_ref):
    o_ref[...] = jnp.max(p_ref[...], axis=1)


@jax.jit
def kernel(img, spx):
    B, C, H, W = img.shape
    img3 = img.reshape(B * C, H, W)
    ranges_per_batch = NW // B
    rows_t = H // ranges_per_batch

    mesh = plsc.VectorSubcoreMesh(
        core_axis_name="c", subcore_axis_name="s", num_cores=NC,
        num_subcores=NS)
    pool = pl.kernel(
        functools.partial(_pool_body, B, C),
        out_type=jax.ShapeDtypeStruct((B * ranges_per_batch * C * K,),
                                      jnp.float32),
        mesh=mesh,
        compiler_params=pltpu.CompilerParams(
            needs_layout_passes=False, use_tc_tiling_on_sc=True),
        scratch_types=(
            [pltpu.VMEM((rows_t, W), jnp.int32)]
            + [pltpu.VMEM((SROWS, W), jnp.float32)
               for _ in range(2 * NCH)]
            + [pltpu.VMEM((L * K,), jnp.float32) for _ in range(NCH)]
            + [pltpu.VMEM((K,), jnp.float32) for _ in range(NCH)]
            + [pltpu.SemaphoreType.DMA for _ in range(2 * NCH)]
        ),
    )
    partial = pool(img3, spx).reshape(B, ranges_per_batch, C, K)

    out = pl.pallas_call(
        _merge_body,
        grid=(B,),
        in_specs=[pl.BlockSpec((1, ranges_per_batch, C, K),
                               lambda i: (i, 0, 0, 0))],
        out_specs=pl.BlockSpec((1, C, K), lambda i: (i, 0, 0)),
        out_shape=jax.ShapeDtypeStruct((B, C, K), jnp.float32),
    )(partial)
    return out
